# baseline (speedup 1.0000x reference)
import functools

import jax
import jax.numpy as jnp
from jax import lax
from jax.experimental import pallas as pl
from jax.experimental.pallas import tpu as pltpu
from jax.experimental.pallas import tpu_sc as plsc

N_NODES = 10000
NP = 10240
D = 128
E = 320000
NC, NS = 2, 16
NW = NC * NS
CHUNK = 32
NCH = E // CHUNK
CPW_LO = NCH // NW
NREM = NCH - CPW_LO * NW
CPW_HI = CPW_LO + 1
ROWS_PT = NP // NS
BLK = 1024
F32 = jnp.float32


def _sc_mesh():
    return plsc.VectorSubcoreMesh(
        core_axis_name="c", subcore_axis_name="s", num_cores=NC, num_subcores=NS
    )


def _deg_body(edges_hbm, out_hbm, locdeg, dbuf):
    wid = lax.axis_index("c") * NS + lax.axis_index("s")
    zeros16 = jnp.zeros((16,), F32)

    def zero_body(i, carry):
        locdeg[pl.ds(i * 16, 16)] = zeros16
        return carry

    lax.fori_loop(0, NP // 16, zero_body, 0)
    base_e = (CPW_LO * wid + jnp.minimum(wid, NREM)) * CHUNK
    pltpu.sync_copy(edges_hbm.at[pl.ds(E + base_e, CPW_LO * CHUNK)], dbuf.at[pl.ds(0, CPW_LO * CHUNK)])

    @pl.when(wid < NREM)
    def _():
        pltpu.sync_copy(
            edges_hbm.at[pl.ds(E + base_e + CPW_LO * CHUNK, CHUNK)],
            dbuf.at[pl.ds(CPW_LO * CHUNK, CHUNK)],
        )

    ones16 = jnp.ones((16,), F32)

    def cnt_body(j, carry):
        idx = dbuf[pl.ds(j * 16, 16)]
        plsc.addupdate_scatter(locdeg, [idx], ones16)
        return carry

    n16 = (CPW_LO * CHUNK) // 16 + jnp.where(wid < NREM, CHUNK // 16, 0)
    lax.fori_loop(0, n16, cnt_body, 0)
    pltpu.sync_copy(locdeg, out_hbm.at[wid])


def _deg_call(edges):
    k = pl.kernel(
        _deg_body,
        out_type=jax.ShapeDtypeStruct((NW, NP), F32),
        mesh=_sc_mesh(),
        scratch_types=[
            pltpu.VMEM((NP,), F32),
            pltpu.VMEM((CPW_HI * CHUNK,), jnp.int32),
        ],
        compiler_params=pltpu.CompilerParams(needs_layout_passes=False),
    )
    return k(edges)


def _agg_body(
    edges_hbm, g_hbm, zeros_hbm, out_hbm, pks, rows, acc, *sems
):
    cid = lax.axis_index("c")
    sid = lax.axis_index("s")
    wid = cid * NS + sid
    base_c = CPW_LO * wid + jnp.minimum(wid, NREM)
    n_w = CPW_LO + jnp.where(wid < NREM, 1, 0)
    gsem = sems[:8]
    isem = sems[8:]

    def idx_issue(k, s):
        off = (base_c + k) * CHUNK
        pltpu.async_copy(edges_hbm.at[pl.ds(off, CHUNK)], pks.at[s, 0], isem[s])
        pltpu.async_copy(edges_hbm.at[pl.ds(E + off, CHUNK)], pks.at[s, 1], isem[s])

    def idx_wait(k, s):
        off = (base_c + k) * CHUNK
        pltpu.make_async_copy(edges_hbm.at[pl.ds(off, CHUNK)], pks.at[s, 0], isem[s]).wait()
        pltpu.make_async_copy(edges_hbm.at[pl.ds(E + off, CHUNK)], pks.at[s, 1], isem[s]).wait()

    for s in range(16):
        idx_issue(s, s)
    for s in range(8):
        idx_wait(s, s)
        pltpu.async_copy(g_hbm.at[pks.at[s, 0]], rows.at[s], gsem[s])
    pltpu.sync_copy(zeros_hbm, acc.at[pl.ds(sid * ROWS_PT, ROWS_PT)])
    plsc.subcore_barrier()

    def body16(j, carry):
        for b16 in range(16):
            c = 16 * j + b16
            r8 = b16 % 8

            @pl.when(c < n_w)
            def _():
                pltpu.make_async_copy(g_hbm.at[pks.at[b16, 0]], rows.at[r8], gsem[r8]).wait()
                pltpu.sync_copy(rows.at[r8], acc.at[pks.at[b16, 1]], add=True)

            @pl.when(c + 16 < n_w)
            def _():
                idx_issue(c + 16, b16)

            @pl.when(c + 8 < n_w)
            def _():
                s8 = (b16 + 8) % 16
                idx_wait(c + 8, s8)
                pltpu.async_copy(g_hbm.at[pks.at[s8, 0]], rows.at[r8], gsem[r8])

        return carry

    lax.fori_loop(0, (CPW_HI + 15) // 16, body16, 0)
    plsc.subcore_barrier()
    pltpu.sync_copy(
        acc.at[pl.ds(sid * ROWS_PT, ROWS_PT)],
        out_hbm.at[cid, pl.ds(sid * ROWS_PT, ROWS_PT)],
    )


def _agg_call(edges, g, zstripe):
    k = pl.kernel(
        _agg_body,
        out_type=jax.ShapeDtypeStruct((NC, NP, D), F32),
        mesh=_sc_mesh(),
        scratch_types=[
            pltpu.VMEM((16, 2, CHUNK), jnp.int32),
            pltpu.VMEM((8, CHUNK, D), F32),
            pltpu.VMEM_SHARED((NP, D), F32),
        ]
        + [pltpu.SemaphoreType.DMA] * 24,
        compiler_params=pltpu.CompilerParams(needs_layout_passes=False),
    )
    return k(edges, g, zstripe)


def _tc_a_body(dega_ref, x_ref, w1_ref, g1_ref):
    deg = 1.0 + jnp.sum(dega_ref[...], axis=0)
    dv = lax.rsqrt(deg)[:, None]
    h = jnp.dot(x_ref[...], w1_ref[...], preferred_element_type=F32)
    g1_ref[...] = dv * h


def _tc_a(dega, xp, W1):
    return pl.pallas_call(
        _tc_a_body,
        grid=(NP // BLK,),
        in_specs=[
            pl.BlockSpec((NW, BLK), lambda i: (0, i)),
            pl.BlockSpec((BLK, D), lambda i: (i, 0)),
            pl.BlockSpec((D, D), lambda i: (0, 0)),
        ],
        out_specs=pl.BlockSpec((BLK, D), lambda i: (i, 0)),
        out_shape=jax.ShapeDtypeStruct((NP, D), F32),
    )(dega, xp, W1)


def _tc_mid_body(p_ref, g_ref, dega_ref, b_ref, w_ref, out_ref):
    dv = lax.rsqrt(1.0 + jnp.sum(dega_ref[...], axis=0))[:, None]
    s = p_ref[0] + p_ref[1] + g_ref[...]
    t = jnp.maximum(dv * s + b_ref[...], 0.0)
    out_ref[...] = dv * jnp.dot(t, w_ref[...], preferred_element_type=F32)


def _tc_final_body(p_ref, g_ref, dega_ref, b_ref, w_ref, bo_ref, out_ref):
    dv = lax.rsqrt(1.0 + jnp.sum(dega_ref[...], axis=0))[:, None]
    s = p_ref[0] + p_ref[1] + g_ref[...]
    t = jnp.maximum(dv * s + b_ref[...], 0.0)
    out_ref[...] = jnp.dot(t, w_ref[...], preferred_element_type=F32) + bo_ref[...]


def _combine_specs():
    in_specs = [
        pl.BlockSpec((NC, BLK, D), lambda i: (0, i, 0)),
        pl.BlockSpec((BLK, D), lambda i: (i, 0)),
        pl.BlockSpec((NW, BLK), lambda i: (0, i)),
        pl.BlockSpec((1, D), lambda i: (0, 0)),
        pl.BlockSpec((D, D), lambda i: (0, 0)),
    ]
    return in_specs


def _tc_mid(p, g, dega, br, W):
    return pl.pallas_call(
        _tc_mid_body,
        grid=(NP // BLK,),
        in_specs=_combine_specs(),
        out_specs=pl.BlockSpec((BLK, D), lambda i: (i, 0)),
        out_shape=jax.ShapeDtypeStruct((NP, D), F32),
    )(p, g, dega, br, W)


def _tc_final(p, g, dega, br, Wo, bor, dout):
    in_specs = [
        pl.BlockSpec((NC, BLK, D), lambda i: (0, i, 0)),
        pl.BlockSpec((BLK, D), lambda i: (i, 0)),
        pl.BlockSpec((NW, BLK), lambda i: (0, i)),
        pl.BlockSpec((1, D), lambda i: (0, 0)),
        pl.BlockSpec((D, dout), lambda i: (0, 0)),
        pl.BlockSpec((1, dout), lambda i: (0, 0)),
    ]
    return pl.pallas_call(
        _tc_final_body,
        grid=(NP // BLK,),
        in_specs=in_specs,
        out_specs=pl.BlockSpec((BLK, dout), lambda i: (i, 0)),
        out_shape=jax.ShapeDtypeStruct((NP, dout), F32),
    )(p, g, dega, br, Wo, bor)


def kernel(x, edge_index, W1, b1, W2, b2, Wout, bout):
    edges = edge_index.astype(jnp.int32).reshape(2 * E)
    zstripe = jnp.zeros((ROWS_PT, D), F32)
    b1r = b1.reshape(1, D)
    b2r = b2.reshape(1, D)
    dout = Wout.shape[1]
    bor = bout.reshape(1, dout)

    dega = _deg_call(edges)
    g1 = _tc_a(dega, x, W1)
    p1 = _agg_call(edges, g1, zstripe)
    g2 = _tc_mid(p1, g1, dega, b1r, W2)
    p2 = _agg_call(edges, g2, zstripe)
    return _tc_final(p2, g2, dega, b2r, Wout, bor, dout)[:N_NODES]


# speedup vs baseline: 1.0076x; 1.0076x over previous
import jax
import jax.numpy as jnp
from jax import lax
from jax.experimental import pallas as pl
from jax.experimental.pallas import tpu as pltpu
from jax.experimental.pallas import tpu_sc as plsc

N_NODES = 10000
NP = 10240
D = 128
E = 320000
NC, NS = 2, 16
NW = NC * NS
CHUNK = 64
NCH = E // CHUNK
CPW_LO = NCH // NW
NREM = NCH - CPW_LO * NW
CPW_HI = CPW_LO + 1
ROWS_PT = NP // NS
BLK = 1024
F32 = jnp.float32


def _sc_mesh():
    return plsc.VectorSubcoreMesh(
        core_axis_name="c", subcore_axis_name="s", num_cores=NC, num_subcores=NS
    )


def _deg_body(edges_hbm, out_hbm, locdeg, dbuf):
    wid = lax.axis_index("c") * NS + lax.axis_index("s")
    zeros16 = jnp.zeros((16,), F32)

    def zero_body(i, carry):
        locdeg[pl.ds(i * 16, 16)] = zeros16
        return carry

    lax.fori_loop(0, NP // 16, zero_body, 0)
    base_e = (CPW_LO * wid + jnp.minimum(wid, NREM)) * CHUNK
    pltpu.sync_copy(edges_hbm.at[pl.ds(E + base_e, CPW_LO * CHUNK)], dbuf.at[pl.ds(0, CPW_LO * CHUNK)])

    @pl.when(wid < NREM)
    def _():
        pltpu.sync_copy(
            edges_hbm.at[pl.ds(E + base_e + CPW_LO * CHUNK, CHUNK)],
            dbuf.at[pl.ds(CPW_LO * CHUNK, CHUNK)],
        )

    ones16 = jnp.ones((16,), F32)

    def cnt_body(j, carry):
        idx = dbuf[pl.ds(j * 16, 16)]
        plsc.addupdate_scatter(locdeg, [idx], ones16)
        return carry

    n16 = (CPW_LO * CHUNK) // 16 + jnp.where(wid < NREM, CHUNK // 16, 0)
    lax.fori_loop(0, n16, cnt_body, 0)
    pltpu.sync_copy(locdeg, out_hbm.at[wid])


def _deg_call(edges):
    k = pl.kernel(
        _deg_body,
        out_type=jax.ShapeDtypeStruct((NW, NP), F32),
        mesh=_sc_mesh(),
        scratch_types=[
            pltpu.VMEM((NP,), F32),
            pltpu.VMEM((CPW_HI * CHUNK,), jnp.int32),
        ],
        compiler_params=pltpu.CompilerParams(needs_layout_passes=False),
    )
    return k(edges)


def _agg_body(
    edges_hbm, g_hbm, zeros_hbm, out_hbm,
    pks, rows, acc, gs0, gs1, gs2, gs3,
    is0, is1, is2, is3, is4, is5, is6, is7,
):
    cid = lax.axis_index("c")
    sid = lax.axis_index("s")
    wid = cid * NS + sid
    base_c = CPW_LO * wid + jnp.minimum(wid, NREM)
    n_w = CPW_LO + jnp.where(wid < NREM, 1, 0)
    gsem = (gs0, gs1, gs2, gs3)
    isem = (is0, is1, is2, is3, is4, is5, is6, is7)

    def idx_issue(k, s):
        off = (base_c + k) * CHUNK
        pltpu.async_copy(edges_hbm.at[pl.ds(off, CHUNK)], pks.at[s, 0], isem[s])
        pltpu.async_copy(edges_hbm.at[pl.ds(E + off, CHUNK)], pks.at[s, 1], isem[s])

    def idx_wait(k, s):
        off = (base_c + k) * CHUNK
        pltpu.make_async_copy(edges_hbm.at[pl.ds(off, CHUNK)], pks.at[s, 0], isem[s]).wait()
        pltpu.make_async_copy(edges_hbm.at[pl.ds(E + off, CHUNK)], pks.at[s, 1], isem[s]).wait()

    for s in range(8):
        idx_issue(s, s)
    for s in range(4):
        idx_wait(s, s)
        pltpu.async_copy(g_hbm.at[pks.at[s, 0]], rows.at[s], gsem[s])
    pltpu.sync_copy(zeros_hbm, acc.at[pl.ds(sid * ROWS_PT, ROWS_PT)])
    plsc.subcore_barrier()

    def body8(j, carry):
        for b8 in range(8):
            c = 8 * j + b8
            r4 = b8 % 4

            @pl.when(c < n_w)
            def _():
                pltpu.make_async_copy(g_hbm.at[pks.at[b8, 0]], rows.at[r4], gsem[r4]).wait()
                pltpu.sync_copy(rows.at[r4], acc.at[pks.at[b8, 1]], add=True)

            @pl.when(c + 8 < n_w)
            def _():
                idx_issue(c + 8, b8)

            @pl.when(c + 4 < n_w)
            def _():
                s4 = (b8 + 4) % 8
                idx_wait(c + 4, s4)
                pltpu.async_copy(g_hbm.at[pks.at[s4, 0]], rows.at[r4], gsem[r4])

        return carry

    lax.fori_loop(0, (CPW_HI + 7) // 8, body8, 0)
    plsc.subcore_barrier()
    pltpu.sync_copy(
        acc.at[pl.ds(sid * ROWS_PT, ROWS_PT)],
        out_hbm.at[cid, pl.ds(sid * ROWS_PT, ROWS_PT)],
    )


def _agg_call(edges, g, zstripe):
    k = pl.kernel(
        _agg_body,
        out_type=jax.ShapeDtypeStruct((NC, NP, D), F32),
        mesh=_sc_mesh(),
        scratch_types=[
            pltpu.VMEM((8, 2, CHUNK), jnp.int32),
            pltpu.VMEM((4, CHUNK, D), F32),
            pltpu.VMEM_SHARED((NP, D), F32),
        ]
        + [pltpu.SemaphoreType.DMA] * 12,
        compiler_params=pltpu.CompilerParams(needs_layout_passes=False),
    )
    return k(edges, g, zstripe)


def _tc_a_body(dega_ref, x_ref, w1_ref, g1_ref):
    deg = 1.0 + jnp.sum(dega_ref[...], axis=0)
    dv = lax.rsqrt(deg)[:, None]
    h = jnp.dot(x_ref[...], w1_ref[...], preferred_element_type=F32)
    g1_ref[...] = dv * h


def _tc_a(dega, xp, W1):
    return pl.pallas_call(
        _tc_a_body,
        grid=(NP // BLK,),
        in_specs=[
            pl.BlockSpec((NW, BLK), lambda i: (0, i)),
            pl.BlockSpec((BLK, D), lambda i: (i, 0)),
            pl.BlockSpec((D, D), lambda i: (0, 0)),
        ],
        out_specs=pl.BlockSpec((BLK, D), lambda i: (i, 0)),
        out_shape=jax.ShapeDtypeStruct((NP, D), F32),
    )(dega, xp, W1)


def _tc_mid_body(p_ref, g_ref, dega_ref, b_ref, w_ref, out_ref):
    dv = lax.rsqrt(1.0 + jnp.sum(dega_ref[...], axis=0))[:, None]
    s = p_ref[0] + p_ref[1] + g_ref[...]
    t = jnp.maximum(dv * s + b_ref[...], 0.0)
    out_ref[...] = dv * jnp.dot(t, w_ref[...], preferred_element_type=F32)


def _tc_final_body(p_ref, g_ref, dega_ref, b_ref, w_ref, bo_ref, out_ref):
    dv = lax.rsqrt(1.0 + jnp.sum(dega_ref[...], axis=0))[:, None]
    s = p_ref[0] + p_ref[1] + g_ref[...]
    t = jnp.maximum(dv * s + b_ref[...], 0.0)
    out_ref[...] = jnp.dot(t, w_ref[...], preferred_element_type=F32) + bo_ref[...]


def _combine_specs():
    in_specs = [
        pl.BlockSpec((NC, BLK, D), lambda i: (0, i, 0)),
        pl.BlockSpec((BLK, D), lambda i: (i, 0)),
        pl.BlockSpec((NW, BLK), lambda i: (0, i)),
        pl.BlockSpec((1, D), lambda i: (0, 0)),
        pl.BlockSpec((D, D), lambda i: (0, 0)),
    ]
    return in_specs


def _tc_mid(p, g, dega, br, W):
    return pl.pallas_call(
        _tc_mid_body,
        grid=(NP // BLK,),
        in_specs=_combine_specs(),
        out_specs=pl.BlockSpec((BLK, D), lambda i: (i, 0)),
        out_shape=jax.ShapeDtypeStruct((NP, D), F32),
    )(p, g, dega, br, W)


def _tc_final(p, g, dega, br, Wo, bor, dout):
    in_specs = [
        pl.BlockSpec((NC, BLK, D), lambda i: (0, i, 0)),
        pl.BlockSpec((BLK, D), lambda i: (i, 0)),
        pl.BlockSpec((NW, BLK), lambda i: (0, i)),
        pl.BlockSpec((1, D), lambda i: (0, 0)),
        pl.BlockSpec((D, dout), lambda i: (0, 0)),
        pl.BlockSpec((1, dout), lambda i: (0, 0)),
    ]
    return pl.pallas_call(
        _tc_final_body,
        grid=(NP // BLK,),
        in_specs=in_specs,
        out_specs=pl.BlockSpec((BLK, dout), lambda i: (i, 0)),
        out_shape=jax.ShapeDtypeStruct((NP, dout), F32),
    )(p, g, dega, br, Wo, bor)


def kernel(x, edge_index, W1, b1, W2, b2, Wout, bout):
    edges = edge_index.astype(jnp.int32).reshape(2 * E)
    zstripe = jnp.zeros((ROWS_PT, D), F32)
    b1r = b1.reshape(1, D)
    b2r = b2.reshape(1, D)
    dout = Wout.shape[1]
    bor = bout.reshape(1, dout)

    dega = _deg_call(edges)
    g1 = _tc_a(dega, x, W1)
    p1 = _agg_call(edges, g1, zstripe)
    g2 = _tc_mid(p1, g1, dega, b1r, W2)
    p2 = _agg_call(edges, g2, zstripe)
    return _tc_final(p2, g2, dega, b2r, Wout, bor, dout)[:N_NODES]
